# baseline (speedup 1.0000x reference)
import functools

import jax
import jax.numpy as jnp
from jax import lax
from jax.experimental import pallas as pl
from jax.experimental.pallas import tpu as pltpu
from jax.experimental.pallas import tpu_sc as plsc

_B = 4096
_F = 26
_V = 100000
_D = 16
_H = 128

_NC = 2
_NS = 16
_NW = _NC * _NS
_SPT = _B // _NW
_CH = 4
_CHF = _CH * _F
_NCH = _SPT // _CH
_PH = 4
_CPP = _NCH // _PH
_SPP = _SPT // _PH


def _sc_body(soidx_hbm, foidx_hbm, xv_hbm, xvt_hbm, sotab_hbm, fotab_hbm,
             so2_hbm, fosum_hbm,
             idx_v, fidx_v, xv_v, xvt_v, rows_v, fo_v, so2_v, fos_v,
             sem_in, sem_f, sem_out, *sem_ph):
    cid = lax.axis_index("c")
    sid = lax.axis_index("s")
    w = sid * _NC + cid

    cps = [
        pltpu.async_copy(soidx_hbm.at[w], idx_v, sem_in),
        pltpu.async_copy(foidx_hbm.at[w], fidx_v, sem_in),
        pltpu.async_copy(xv_hbm.at[w], xv_v, sem_in),
        pltpu.async_copy(xvt_hbm.at[w], xvt_v, sem_in),
    ]
    for cp in cps:
        cp.wait()

    fo_cps = []
    for f in range(_F):
        fo_cps.append(pltpu.async_copy(
            fotab_hbm.at[fidx_v.at[f]],
            fo_v.at[pl.ds(f * _SPT, _SPT)],
            sem_f,
        ))

    def fire_phase(p):
        cps = []
        for ci in range(p * _CPP, (p + 1) * _CPP):
            cps.append(pltpu.async_copy(
                sotab_hbm.at[idx_v.at[ci]],
                rows_v.at[pl.ds(ci * _CHF, _CHF)],
                sem_ph[p],
            ))
        return cps

    def sample_body(j, _):
        xlo = xv_v[2 * j]
        xhi = xv_v[2 * j + 1]
        base = j * _F
        s_acc = jnp.zeros((16,), jnp.float32)
        q_acc = jnp.zeros((16,), jnp.float32)
        for f in range(_F):
            src = xlo if f < 16 else xhi
            xvb = jnp.take_along_axis(
                src, jnp.full((16,), f % 16, jnp.int32), axis=0,
                mode="promise_in_bounds")
            e = rows_v[base + f] * xvb
            s_acc = s_acc + e
            q_acc = q_acc + e * e
        so2_v[j] = (s_acc * s_acc - q_acc) * 0.5
        return _

    ph_cps = [fire_phase(0)]
    for p in range(_PH):
        if p + 1 < _PH:
            ph_cps.append(fire_phase(p + 1))
        for cp in ph_cps[p]:
            cp.wait()
        lax.fori_loop(p * _SPP, (p + 1) * _SPP, sample_body, 0)

    for cp in fo_cps:
        cp.wait()

    def fo_group(v, _):
        off = v * 16
        acc = jnp.zeros((16,), jnp.float32)
        for f in range(_F):
            acc = acc + (fo_v[pl.ds(f * _SPT + off, 16)]
                         * xvt_v[pl.ds(f * _SPT + off, 16)])
        fos_v[pl.ds(off, 16)] = acc
        return _

    lax.fori_loop(0, _SPT // 16, fo_group, 0)

    out_cps = [
        pltpu.async_copy(so2_v, so2_hbm.at[pl.ds(w * _SPT, _SPT)], sem_out),
        pltpu.async_copy(fos_v, fosum_hbm.at[pl.ds(w * _SPT, _SPT)], sem_out),
    ]
    for cp in out_cps:
        cp.wait()


@functools.partial(
    pl.kernel,
    out_type=(
        jax.ShapeDtypeStruct((_B, _D), jnp.float32),
        jax.ShapeDtypeStruct((_B,), jnp.float32),
    ),
    mesh=plsc.VectorSubcoreMesh(core_axis_name="c", subcore_axis_name="s"),
    compiler_params=pltpu.CompilerParams(use_tc_tiling_on_sc=False),
    scratch_types=(
        [
            pltpu.VMEM((_NCH, _CHF), jnp.int32),
            pltpu.VMEM((_F, _SPT), jnp.int32),
            pltpu.VMEM((2 * _SPT, 16), jnp.float32),
            pltpu.VMEM((_F * _SPT,), jnp.float32),
            pltpu.VMEM((_SPT * _F, _D), jnp.float32),
            pltpu.VMEM((_F * _SPT,), jnp.float32),
            pltpu.VMEM((_SPT, _D), jnp.float32),
            pltpu.VMEM((_SPT,), jnp.float32),
            pltpu.SemaphoreType.DMA,
            pltpu.SemaphoreType.DMA,
            pltpu.SemaphoreType.DMA,
        ]
        + [pltpu.SemaphoreType.DMA for _ in range(_PH)]
    ),
)
def _sc_gather_fm(soidx, foidx, xvs, xvt, sotab, fotab, so2, fosum, *rest):
    _sc_body(soidx, foidx, xvs, xvt, sotab, fotab, so2, fosum, *rest)


def _tc_body(so2_ref, fos_ref, w0_ref, b0_ref, w1_ref, b1_ref, bias_ref,
             out_ref):
    x = so2_ref[...]
    h = jnp.dot(x, w0_ref[...], preferred_element_type=jnp.float32)
    h = jnp.maximum(h + b0_ref[...], 0.0)
    h = jnp.dot(h, w1_ref[...], preferred_element_type=jnp.float32)
    h = jnp.maximum(h + b1_ref[...], 0.0)
    out_ref[...] = bias_ref[0, 0] + fos_ref[...] + jnp.sum(h, axis=1)


def _tc_mlp(so2, fosum, W0, b0, W1, b1, bias2d):
    return pl.pallas_call(
        _tc_body,
        out_shape=jax.ShapeDtypeStruct((_B,), jnp.float32),
        in_specs=[
            pl.BlockSpec(memory_space=pltpu.VMEM),
            pl.BlockSpec(memory_space=pltpu.VMEM),
            pl.BlockSpec(memory_space=pltpu.VMEM),
            pl.BlockSpec(memory_space=pltpu.VMEM),
            pl.BlockSpec(memory_space=pltpu.VMEM),
            pl.BlockSpec(memory_space=pltpu.VMEM),
            pl.BlockSpec(memory_space=pltpu.SMEM),
        ],
        out_specs=pl.BlockSpec(memory_space=pltpu.VMEM),
    )(so2, fosum, W0, b0, W1, b1, bias2d)


def kernel(Xi, Xv, fo_emb, so_emb, W0, b0, W1, b1, b):
    idx = Xi[:, :, 0].astype(jnp.int32)
    flat = idx + (jnp.arange(_F, dtype=jnp.int32) * _V)[None, :]
    soidx = flat.reshape(_NW, _NCH, _CHF)
    foidx = flat.reshape(_NW, _SPT, _F).transpose(0, 2, 1)
    xvp = jnp.pad(Xv, ((0, 0), (0, 32 - _F)))
    xvs = xvp.reshape(_NW, 2 * _SPT, 16)
    xvt = Xv.reshape(_NW, _SPT, _F).transpose(0, 2, 1).reshape(_NW, _F * _SPT)
    sotab = so_emb.reshape(_F * _V, _D)
    fotab = fo_emb.reshape(_F * _V)
    so2, fosum = _sc_gather_fm(soidx, foidx, xvs, xvt, sotab, fotab)
    return _tc_mlp(so2, fosum, W0, b0, W1, b1,
                   jnp.reshape(b.astype(jnp.float32), (1, 1)))
